# baseline (speedup 1.0000x reference)
import functools

import jax
import jax.numpy as jnp
from jax import lax
from jax.experimental import pallas as pl
from jax.experimental.pallas import tpu as pltpu
from jax.experimental.pallas import tpu_sc as plsc

VOCAB = 150528
D = 1024
B = 4
S = 2048
MASK_TOKEN = 150001
BOS_TOKEN = 150004

N_IDS = B * S
NC, NS = 2, 16
NW = NC * NS
IDS_PER_W = N_IDS // NW
CHUNK = 32
N_CHUNKS = IDS_PER_W // CHUNK


def _sc_gather(weight, ids_flat):
    mesh = plsc.VectorSubcoreMesh(core_axis_name="c", subcore_axis_name="s")

    @functools.partial(
        pl.kernel,
        out_type=jax.ShapeDtypeStruct((N_IDS, D), jnp.float32),
        mesh=mesh,
        scratch_types=[
            pltpu.VMEM((IDS_PER_W,), jnp.int32),
            pltpu.VMEM((CHUNK, D), jnp.float32),
            pltpu.SemaphoreType.DMA,
        ],
    )
    def gather_kernel(w_hbm, i_hbm, o_hbm, idx_v, rows_v, sem):
        wid = lax.axis_index("s") * NC + lax.axis_index("c")
        base = wid * IDS_PER_W
        pltpu.sync_copy(i_hbm.at[pl.ds(base, IDS_PER_W)], idx_v)
        for c in range(N_CHUNKS):
            pltpu.async_copy(
                w_hbm.at[idx_v.at[pl.ds(c * CHUNK, CHUNK)]], rows_v, sem
            ).wait()
            pltpu.sync_copy(rows_v, o_hbm.at[pl.ds(base + c * CHUNK, CHUNK)])

    return gather_kernel(weight, ids_flat)


MASK_TILE = 256


def _mask_pos_kernel(ids_ref, mask_ref, pos_ref):
    t = pl.program_id(1)
    ids = ids_ref[0]
    j2 = lax.broadcasted_iota(jnp.int32, (1, S), 1)
    ctx = jnp.min(jnp.where(ids == BOS_TOKEN, j2, S))
    mpos = jnp.min(jnp.where(ids == MASK_TOKEN, j2, S))

    i = lax.broadcasted_iota(jnp.int32, (MASK_TILE, S), 0) + t * MASK_TILE
    j = lax.broadcasted_iota(jnp.int32, (MASK_TILE, S), 1)
    mask_ref[0, 0, :, :] = (j > i) & (j >= ctx)

    @pl.when(t == 0)
    def _():
        pos_ref[:, 0, :] = jnp.where(j2 >= ctx, mpos, j2)
        pos_ref[:, 1, :] = jnp.where(j2 < ctx, 0, j2 - ctx + 1)


def _tc_mask_pos(input_ids):
    return pl.pallas_call(
        _mask_pos_kernel,
        grid=(B, S // MASK_TILE),
        in_specs=[pl.BlockSpec((1, 1, S), lambda b, t: (b, 0, 0))],
        out_specs=[
            pl.BlockSpec((1, 1, MASK_TILE, S), lambda b, t: (b, 0, t, 0)),
            pl.BlockSpec((1, 2, S), lambda b, t: (b, 0, 0)),
        ],
        out_shape=[
            jax.ShapeDtypeStruct((B, 1, S, S), jnp.bool_),
            jax.ShapeDtypeStruct((B, 2, S), jnp.int32),
        ],
    )(input_ids.reshape(B, 1, S))


def kernel(input_ids, labels, weight):
    ids_flat = input_ids.T.reshape(N_IDS)
    hidden = _sc_gather(weight, ids_flat).reshape(S, B, D)
    attention_mask, position_ids = _tc_mask_pos(input_ids)
    return hidden, position_ids, attention_mask, labels
